# Initial kernel scaffold; baseline (speedup 1.0000x reference)
import jax
import jax.numpy as jnp
from jax.experimental import pallas as pl


def kernel(x, edge_index, W0, b0, W1, b1):
    raise NotImplementedError("write your pallas kernel here")


# speedup vs baseline: 5.2982x; 5.2982x over previous
import functools

import jax
import jax.numpy as jnp
from jax import lax
from jax.experimental import pallas as pl
from jax.experimental.pallas import tpu as pltpu
from jax.experimental.pallas import tpu_sc as plsc

N = 10000
E = 320000
D = 128
NC = 2
NS = 16
NW = NC * NS
L = 16
NPAD = 10240
CH = 128
NCH = 80
EPW = NCH * CH
E_PAD = EPW * NW
RPT = NPAD // NS

_mesh = plsc.VectorSubcoreMesh(
    core_axis_name="c", subcore_axis_name="s", num_cores=NC, num_subcores=NS)

_f32 = jnp.float32


def _zero_fill_1d(ref, n):
    def body(i, _):
        ref[pl.ds(i * L, L)] = jnp.zeros((L,), _f32)
        return _
    lax.fori_loop(0, n // L, body, None)


@functools.partial(
    pl.kernel,
    out_type=(jax.ShapeDtypeStruct((NC, NPAD), _f32),
              jax.ShapeDtypeStruct((NC, NPAD), _f32)),
    mesh=_mesh,
    scratch_types=[
        pltpu.VMEM((NCH, CH), jnp.int32),
        pltpu.VMEM((CH,), _f32),
        pltpu.VMEM((RPT,), _f32),
        pltpu.VMEM_SHARED((NPAD,), _f32),
        pltpu.VMEM_SHARED((NPAD,), _f32),
    ],
)
def _deg_kernel(srcr, dstr, out_src, out_dst, idx, ones, zb, dsrc_sh, ddst_sh):
    cid = lax.axis_index("c")
    sid = lax.axis_index("s")
    wid = sid * NC + cid

    def fill_ones(i, _):
        ones[pl.ds(i * L, L)] = jnp.ones((L,), _f32)
        return _
    lax.fori_loop(0, CH // L, fill_ones, None)
    _zero_fill_1d(zb, RPT)
    pltpu.sync_copy(zb, dsrc_sh.at[pl.ds(sid * RPT, RPT)])
    pltpu.sync_copy(zb, ddst_sh.at[pl.ds(sid * RPT, RPT)])
    plsc.subcore_barrier()

    pltpu.sync_copy(srcr.at[wid], idx)

    def body_s(j, _):
        pltpu.sync_copy(ones, dsrc_sh.at[idx.at[j]], add=True)
        return _
    lax.fori_loop(0, NCH, body_s, None)

    pltpu.sync_copy(dstr.at[wid], idx)

    def body_d(j, _):
        pltpu.sync_copy(ones, ddst_sh.at[idx.at[j]], add=True)
        return _
    lax.fori_loop(0, NCH, body_d, None)

    plsc.subcore_barrier()
    sl = pl.ds(sid * RPT, RPT)
    pltpu.sync_copy(dsrc_sh.at[sl], out_src.at[cid].at[sl])
    pltpu.sync_copy(ddst_sh.at[sl], out_dst.at[cid].at[sl])


@functools.partial(
    pl.kernel,
    out_type=jax.ShapeDtypeStruct((NC, NPAD, D), _f32),
    mesh=_mesh,
    scratch_types=[
        pltpu.VMEM((NCH, CH), jnp.int32),
        pltpu.VMEM((NCH, CH), jnp.int32),
        pltpu.VMEM((CH, D), _f32),
        pltpu.VMEM_SHARED((NPAD, D), _f32),
    ],
)
def _agg_kernel(srcr, dstr, y, out, idx_s, idx_d, rows, agg_sh):
    cid = lax.axis_index("c")
    sid = lax.axis_index("s")
    wid = sid * NC + cid

    def zfill(k, _):
        rows[k // (D // L), pl.ds((k % (D // L)) * L, L)] = jnp.zeros((L,), _f32)
        return _
    lax.fori_loop(0, CH * (D // L), zfill, None)
    for t in range(RPT // CH):
        pltpu.sync_copy(rows, agg_sh.at[pl.ds(sid * RPT + t * CH, CH)])
    plsc.subcore_barrier()

    pltpu.sync_copy(srcr.at[wid], idx_s)
    pltpu.sync_copy(dstr.at[wid], idx_d)

    def body(j, _):
        pltpu.sync_copy(y.at[idx_s.at[j]], rows)
        pltpu.sync_copy(rows, agg_sh.at[idx_d.at[j]], add=True)
        return _
    lax.fori_loop(0, NCH, body, None)

    plsc.subcore_barrier()
    sl = pl.ds(sid * RPT, RPT)
    pltpu.sync_copy(agg_sh.at[sl], out.at[cid].at[sl])


@functools.partial(
    pl.kernel,
    out_type=jax.ShapeDtypeStruct((NC, NPAD), _f32),
    mesh=_mesh,
    scratch_types=[
        pltpu.VMEM((NCH, CH), jnp.int32),
        pltpu.VMEM((NCH, CH), jnp.int32),
        pltpu.VMEM((CH,), _f32),
        pltpu.VMEM((RPT,), _f32),
        pltpu.VMEM_SHARED((NPAD,), _f32),
    ],
)
def _seg2_kernel(srcr, dstr, g, out, idx_s, idx_d, vals, zb, s_sh):
    cid = lax.axis_index("c")
    sid = lax.axis_index("s")
    wid = sid * NC + cid

    _zero_fill_1d(zb, RPT)
    pltpu.sync_copy(zb, s_sh.at[pl.ds(sid * RPT, RPT)])
    plsc.subcore_barrier()

    pltpu.sync_copy(srcr.at[wid], idx_s)
    pltpu.sync_copy(dstr.at[wid], idx_d)

    def body(j, _):
        pltpu.sync_copy(g.at[idx_s.at[j]], vals)
        pltpu.sync_copy(vals, s_sh.at[idx_d.at[j]], add=True)
        return _
    lax.fori_loop(0, NCH, body, None)

    plsc.subcore_barrier()
    sl = pl.ds(sid * RPT, RPT)
    pltpu.sync_copy(s_sh.at[sl], out.at[cid].at[sl])


def _dense1_body(xp_ref, w0_ref, ds_ref, dd_ref, y_ref, ns_ref, nd_ref):
    ns = lax.rsqrt(jnp.maximum(ds_ref[0] + ds_ref[1], 1.0))
    nd = lax.rsqrt(jnp.maximum(dd_ref[0] + dd_ref[1], 1.0))
    ns_ref[...] = ns
    nd_ref[...] = nd
    y = jnp.dot(xp_ref[...], w0_ref[...], preferred_element_type=_f32)
    y_ref[...] = y * ns[:, None]


def _dense2_body(p_ref, b0_ref, w1_ref, nd_ref, ns_ref, g_ref):
    z = (p_ref[0] + p_ref[1]) * nd_ref[...][:, None] + b0_ref[...][None, :]
    h1 = jnp.maximum(z, 0.0)
    w1m = jnp.mean(w1_ref[...], axis=1)
    g_ref[...] = jnp.sum(h1 * w1m[None, :], axis=1) * ns_ref[...]


def _dense3_body(s_ref, nd_ref, b1_ref, o_ref):
    o_ref[...] = (s_ref[0] + s_ref[1]) * nd_ref[...] + jnp.mean(b1_ref[...])


_dense1 = pl.pallas_call(
    _dense1_body,
    out_shape=(jax.ShapeDtypeStruct((NPAD, D), _f32),
               jax.ShapeDtypeStruct((NPAD,), _f32),
               jax.ShapeDtypeStruct((NPAD,), _f32)))

_dense2 = pl.pallas_call(
    _dense2_body,
    out_shape=jax.ShapeDtypeStruct((NPAD,), _f32))

_dense3 = pl.pallas_call(
    _dense3_body,
    out_shape=jax.ShapeDtypeStruct((NPAD,), _f32))


def kernel(x, edge_index, W0, b0, W1, b1):
    src = edge_index[0].astype(jnp.int32)
    dst = edge_index[1].astype(jnp.int32)
    pad = jnp.full((E_PAD - E,), NPAD - 1, jnp.int32)
    srcr = jnp.concatenate([src, pad]).reshape(NW, NCH, CH)
    dstr = jnp.concatenate([dst, pad]).reshape(NW, NCH, CH)
    xp = jnp.pad(x, ((0, NPAD - N), (0, 0)))

    deg_s, deg_d = _deg_kernel(srcr, dstr)
    y, ns, nd = _dense1(xp, W0, deg_s, deg_d)
    p = _agg_kernel(srcr, dstr, y)
    g = _dense2(p, b0, W1, nd, ns)
    s = _seg2_kernel(srcr, dstr, g)
    out = _dense3(s, nd, b1)
    return out[:N]


# speedup vs baseline: 5.6103x; 1.0589x over previous
import functools

import jax
import jax.numpy as jnp
from jax import lax
from jax.experimental import pallas as pl
from jax.experimental.pallas import tpu as pltpu
from jax.experimental.pallas import tpu_sc as plsc

N = 10000
E = 320000
D = 128
NC = 2
NS = 16
NW = NC * NS
L = 16
NPAD = 10240
CH = 128
NCH = 80
EPW = NCH * CH
E_PAD = EPW * NW
RPT = NPAD // NS

_mesh = plsc.VectorSubcoreMesh(
    core_axis_name="c", subcore_axis_name="s", num_cores=NC, num_subcores=NS)

_f32 = jnp.float32


def _zero_fill_1d(ref, n):
    def body(i, _):
        ref[pl.ds(i * L, L)] = jnp.zeros((L,), _f32)
        return _
    lax.fori_loop(0, n // L, body, None)


@functools.partial(
    pl.kernel,
    out_type=(jax.ShapeDtypeStruct((NC, NPAD), _f32),
              jax.ShapeDtypeStruct((NC, NPAD), _f32)),
    mesh=_mesh,
    scratch_types=[
        pltpu.VMEM((NCH, CH), jnp.int32),
        pltpu.VMEM((CH,), _f32),
        pltpu.VMEM((RPT,), _f32),
        pltpu.VMEM_SHARED((NPAD,), _f32),
        pltpu.VMEM_SHARED((NPAD,), _f32),
    ],
)
def _deg_kernel(srcr, dstr, out_src, out_dst, idx, ones, zb, dsrc_sh, ddst_sh):
    cid = lax.axis_index("c")
    sid = lax.axis_index("s")
    wid = sid * NC + cid

    def fill_ones(i, _):
        ones[pl.ds(i * L, L)] = jnp.ones((L,), _f32)
        return _
    lax.fori_loop(0, CH // L, fill_ones, None)
    _zero_fill_1d(zb, RPT)
    pltpu.sync_copy(zb, dsrc_sh.at[pl.ds(sid * RPT, RPT)])
    pltpu.sync_copy(zb, ddst_sh.at[pl.ds(sid * RPT, RPT)])
    plsc.subcore_barrier()

    pltpu.sync_copy(srcr.at[wid], idx)

    def body_s(j, _):
        pltpu.sync_copy(ones, dsrc_sh.at[idx.at[j]], add=True)
        return _
    lax.fori_loop(0, NCH, body_s, None)

    pltpu.sync_copy(dstr.at[wid], idx)

    def body_d(j, _):
        pltpu.sync_copy(ones, ddst_sh.at[idx.at[j]], add=True)
        return _
    lax.fori_loop(0, NCH, body_d, None)

    plsc.subcore_barrier()
    sl = pl.ds(sid * RPT, RPT)
    pltpu.sync_copy(dsrc_sh.at[sl], out_src.at[cid].at[sl])
    pltpu.sync_copy(ddst_sh.at[sl], out_dst.at[cid].at[sl])


@functools.partial(
    pl.kernel,
    out_type=jax.ShapeDtypeStruct((NC, NPAD, D), _f32),
    mesh=_mesh,
    scratch_types=[
        pltpu.VMEM((CH,), jnp.int32),
        pltpu.VMEM((CH,), jnp.int32),
        pltpu.VMEM((CH,), jnp.int32),
        pltpu.VMEM((CH,), jnp.int32),
        pltpu.VMEM((CH, D), _f32),
        pltpu.VMEM((CH, D), _f32),
        pltpu.SemaphoreType.DMA,
        pltpu.SemaphoreType.DMA,
        pltpu.SemaphoreType.DMA,
        pltpu.SemaphoreType.DMA,
        pltpu.VMEM_SHARED((NPAD, D), _f32),
    ],
)
def _agg_kernel(srcr, dstr, y, out, is0, is1, id0, id1, rows0, rows1,
                sem0, sem1, semi0, semi1, agg_sh):
    cid = lax.axis_index("c")
    sid = lax.axis_index("s")
    wid = sid * NC + cid

    def zfill(k, _):
        rows0[k // (D // L), pl.ds((k % (D // L)) * L, L)] = jnp.zeros((L,), _f32)
        return _
    lax.fori_loop(0, CH * (D // L), zfill, None)
    for t in range(RPT // CH):
        pltpu.sync_copy(rows0, agg_sh.at[pl.ds(sid * RPT + t * CH, CH)])
    plsc.subcore_barrier()

    def _idx_load(j, isb, idb, semi):
        pltpu.async_copy(srcr.at[wid].at[j], isb, semi)
        pltpu.async_copy(dstr.at[wid].at[j], idb, semi)

    def _idx_wait(isb, idb, semi):
        pltpu.make_async_copy(srcr.at[wid].at[0], isb, semi).wait()
        pltpu.make_async_copy(dstr.at[wid].at[0], idb, semi).wait()

    def _gather_wait(rowsb, sem):
        pltpu.make_async_copy(y.at[pl.ds(0, CH)], rowsb, sem).wait()

    _idx_load(0, is0, id0, semi0)
    _idx_wait(is0, id0, semi0)
    _idx_load(1, is1, id1, semi1)
    pltpu.async_copy(y.at[is0], rows0, sem0)

    def body(i, _):
        j0 = 2 * i
        _gather_wait(rows0, sem0)
        _idx_wait(is1, id1, semi1)
        pltpu.async_copy(y.at[is1], rows1, sem1)
        pltpu.sync_copy(rows0, agg_sh.at[id0], add=True)
        _idx_load(jnp.minimum(j0 + 2, NCH - 1), is0, id0, semi0)
        _gather_wait(rows1, sem1)
        _idx_wait(is0, id0, semi0)
        pltpu.async_copy(y.at[is0], rows0, sem0)
        pltpu.sync_copy(rows1, agg_sh.at[id1], add=True)
        _idx_load(jnp.minimum(j0 + 3, NCH - 1), is1, id1, semi1)
        return _
    lax.fori_loop(0, NCH // 2, body, None)
    _gather_wait(rows0, sem0)
    _idx_wait(is1, id1, semi1)

    plsc.subcore_barrier()
    sl = pl.ds(sid * RPT, RPT)
    pltpu.sync_copy(agg_sh.at[sl], out.at[cid].at[sl])


@functools.partial(
    pl.kernel,
    out_type=jax.ShapeDtypeStruct((NC, NPAD), _f32),
    mesh=_mesh,
    scratch_types=[
        pltpu.VMEM((NCH, CH), jnp.int32),
        pltpu.VMEM((NCH, CH), jnp.int32),
        pltpu.VMEM((CH,), _f32),
        pltpu.VMEM((RPT,), _f32),
        pltpu.VMEM_SHARED((NPAD,), _f32),
    ],
)
def _seg2_kernel(srcr, dstr, g, out, idx_s, idx_d, vals, zb, s_sh):
    cid = lax.axis_index("c")
    sid = lax.axis_index("s")
    wid = sid * NC + cid

    _zero_fill_1d(zb, RPT)
    pltpu.sync_copy(zb, s_sh.at[pl.ds(sid * RPT, RPT)])
    plsc.subcore_barrier()

    pltpu.sync_copy(srcr.at[wid], idx_s)
    pltpu.sync_copy(dstr.at[wid], idx_d)

    def body(j, _):
        pltpu.sync_copy(g.at[idx_s.at[j]], vals)
        pltpu.sync_copy(vals, s_sh.at[idx_d.at[j]], add=True)
        return _
    lax.fori_loop(0, NCH, body, None)

    plsc.subcore_barrier()
    sl = pl.ds(sid * RPT, RPT)
    pltpu.sync_copy(s_sh.at[sl], out.at[cid].at[sl])


def _dense1_body(xp_ref, w0_ref, ds_ref, dd_ref, y_ref, ns_ref, nd_ref):
    ns = lax.rsqrt(jnp.maximum(ds_ref[0] + ds_ref[1], 1.0))
    nd = lax.rsqrt(jnp.maximum(dd_ref[0] + dd_ref[1], 1.0))
    ns_ref[...] = ns
    nd_ref[...] = nd
    y = jnp.dot(xp_ref[...], w0_ref[...], preferred_element_type=_f32)
    y_ref[...] = y * ns[:, None]


def _dense2_body(p_ref, b0_ref, w1_ref, nd_ref, ns_ref, g_ref):
    z = (p_ref[0] + p_ref[1]) * nd_ref[...][:, None] + b0_ref[...][None, :]
    h1 = jnp.maximum(z, 0.0)
    w1m = jnp.mean(w1_ref[...], axis=1)
    g_ref[...] = jnp.sum(h1 * w1m[None, :], axis=1) * ns_ref[...]


def _dense3_body(s_ref, nd_ref, b1_ref, o_ref):
    o_ref[...] = (s_ref[0] + s_ref[1]) * nd_ref[...] + jnp.mean(b1_ref[...])


_dense1 = pl.pallas_call(
    _dense1_body,
    out_shape=(jax.ShapeDtypeStruct((NPAD, D), _f32),
               jax.ShapeDtypeStruct((NPAD,), _f32),
               jax.ShapeDtypeStruct((NPAD,), _f32)))

_dense2 = pl.pallas_call(
    _dense2_body,
    out_shape=jax.ShapeDtypeStruct((NPAD,), _f32))

_dense3 = pl.pallas_call(
    _dense3_body,
    out_shape=jax.ShapeDtypeStruct((NPAD,), _f32))


def kernel(x, edge_index, W0, b0, W1, b1):
    src = edge_index[0].astype(jnp.int32)
    dst = edge_index[1].astype(jnp.int32)
    pad = jnp.full((E_PAD - E,), NPAD - 1, jnp.int32)
    srcr = jnp.concatenate([src, pad]).reshape(NW, NCH, CH)
    dstr = jnp.concatenate([dst, pad]).reshape(NW, NCH, CH)
    xp = jnp.pad(x, ((0, NPAD - N), (0, 0)))

    deg_s, deg_d = _deg_kernel(srcr, dstr)
    y, ns, nd = _dense1(xp, W0, deg_s, deg_d)
    p = _agg_kernel(srcr, dstr, y)
    g = _dense2(p, b0, W1, nd, ns)
    s = _seg2_kernel(srcr, dstr, g)
    out = _dense3(s, nd, b1)
    return out[:N]
